# Initial kernel scaffold; baseline (speedup 1.0000x reference)
import jax
import jax.numpy as jnp
from jax.experimental import pallas as pl


def kernel(atom_ids, atom_features, table, Wf, bf, Wc, bc):
    raise NotImplementedError("write your pallas kernel here")


# speedup vs baseline: 1.0010x; 1.0010x over previous
import functools

import jax
import jax.numpy as jnp
from jax import lax
from jax.experimental import pallas as pl
from jax.experimental.pallas import tpu as pltpu
from jax.experimental.pallas import tpu_sc as plsc

D = 128
NF = 4
LANES = 16
NC, NS = 2, 16
NW = NC * NS


def _prep_body(table_ref, wf_ref, bfc_ref, wc_ref, tbl_out_ref, wfc_out_ref):
    wc1 = wc_ref[:D, :]
    wc2 = wc_ref[D:, :]
    bias = (
        jnp.dot(bfc_ref[0:1, :], wc2, preferred_element_type=jnp.float32)
        + bfc_ref[1:2, :]
    )
    tbl_out_ref[...] = (
        jnp.dot(table_ref[...], wc1, preferred_element_type=jnp.float32) + bias
    )
    wfc_out_ref[...] = jnp.dot(wf_ref[...], wc2, preferred_element_type=jnp.float32)


def _prep(table, Wf, bfc, Wc):
    num_atoms = table.shape[0]
    return pl.pallas_call(
        _prep_body,
        out_shape=[
            jax.ShapeDtypeStruct((num_atoms, D), jnp.float32),
            jax.ShapeDtypeStruct((NF, D), jnp.float32),
        ],
    )(table, Wf, bfc, Wc)


@functools.partial(jax.jit, static_argnames=("num_atoms", "n_rows", "chunk"))
def _sc_lookup(ids, feat, tbl, wfc, *, num_atoms, n_rows, chunk):
    rows_per_w = n_rows // NW
    n_chunks = rows_per_w // chunk
    mesh = plsc.VectorSubcoreMesh(
        core_axis_name="c", subcore_axis_name="s", num_cores=NC, num_subcores=NS
    )

    @functools.partial(
        pl.kernel,
        out_type=jax.ShapeDtypeStruct((n_rows, D), jnp.float32),
        mesh=mesh,
        compiler_params=pltpu.CompilerParams(needs_layout_passes=False),
        scratch_types=[
            pltpu.VMEM((num_atoms * D,), jnp.float32),
            pltpu.VMEM((NF, D), jnp.float32),
            pltpu.VMEM((chunk + LANES,), jnp.int32),
            pltpu.VMEM((chunk * NF + LANES,), jnp.float32),
            pltpu.VMEM((chunk, D), jnp.float32),
        ],
    )
    def body(ids_hbm, feat_hbm, tbl_hbm, wfc_hbm, out_hbm, tbl_v, wfc_v, ids_v, feat_v, out_v):
        wid = lax.axis_index("s") * NC + lax.axis_index("c")
        base0 = wid * rows_per_w
        pltpu.sync_copy(tbl_hbm, tbl_v)
        pltpu.sync_copy(wfc_hbm, wfc_v)
        cols = [lax.iota(jnp.int32, LANES) + LANES * j for j in range(D // LANES)]
        w = [
            [wfc_v[k, pl.ds(LANES * j, LANES)] for j in range(D // LANES)]
            for k in range(NF)
        ]

        def chunk_body(it, carry):
            base = base0 + it * chunk
            pltpu.sync_copy(ids_hbm.at[pl.ds(base, chunk)], ids_v.at[pl.ds(0, chunk)])
            pltpu.sync_copy(
                feat_hbm.at[pl.ds(base * NF, chunk * NF)],
                feat_v.at[pl.ds(0, chunk * NF)],
            )

            def row_body(r, rcarry):
                rowbase = jnp.full((LANES,), ids_v[pl.ds(r, LANES)][0] * D, jnp.int32)
                fvec = feat_v[pl.ds(r * NF, LANES)]
                f0 = fvec[0]
                f1 = fvec[1]
                f2 = fvec[2]
                f3 = fvec[3]
                for j in range(D // LANES):
                    g = plsc.load_gather(tbl_v, [rowbase + cols[j]])
                    acc = g + f0 * w[0][j] + f1 * w[1][j] + f2 * w[2][j] + f3 * w[3][j]
                    out_v[r, pl.ds(LANES * j, LANES)] = acc
                return rcarry

            lax.fori_loop(0, chunk, row_body, 0)
            pltpu.sync_copy(out_v, out_hbm.at[pl.ds(base, chunk)])
            return carry

        lax.fori_loop(0, n_chunks, chunk_body, 0)

    return body(ids, feat, tbl, wfc)


def kernel(atom_ids, atom_features, table, Wf, bf, Wc, bc):
    B, L = atom_ids.shape
    n_rows = B * L
    ids = atom_ids.reshape(n_rows).astype(jnp.int32)
    feat = atom_features.reshape(n_rows * NF)
    bfc = jnp.stack([bf, bc])
    tbl, wfc = _prep(table, Wf, bfc, Wc)
    chunk = 256
    out = _sc_lookup(
        ids, feat, tbl.reshape(-1), wfc,
        num_atoms=table.shape[0], n_rows=n_rows, chunk=chunk,
    )
    return out.reshape(B, L, D)


# speedup vs baseline: 1.7291x; 1.7274x over previous
import functools

import jax
import jax.numpy as jnp
from jax import lax
from jax.experimental import pallas as pl
from jax.experimental.pallas import tpu as pltpu
from jax.experimental.pallas import tpu_sc as plsc

D = 128
NF = 4
LANES = 16
NC, NS = 2, 16
NW = NC * NS


def _prep_body(table_ref, wf_ref, bfc_ref, wc_ref, tbl_out_ref, wfc_out_ref):
    wc1 = wc_ref[:D, :]
    wc2 = wc_ref[D:, :]
    bias = (
        jnp.dot(bfc_ref[0:1, :], wc2, preferred_element_type=jnp.float32)
        + bfc_ref[1:2, :]
    )
    tbl_out_ref[...] = (
        jnp.dot(table_ref[...], wc1, preferred_element_type=jnp.float32) + bias
    )
    wfc_out_ref[...] = jnp.dot(wf_ref[...], wc2, preferred_element_type=jnp.float32)


def _prep(table, Wf, bfc, Wc):
    num_atoms = table.shape[0]
    return pl.pallas_call(
        _prep_body,
        out_shape=[
            jax.ShapeDtypeStruct((num_atoms, D), jnp.float32),
            jax.ShapeDtypeStruct((NF, D), jnp.float32),
        ],
    )(table, Wf, bfc, Wc)


@functools.partial(jax.jit, static_argnames=("num_atoms", "n_rows", "chunk"))
def _sc_lookup(ids, feat, tbl, wfc, *, num_atoms, n_rows, chunk):
    rows_per_w = n_rows // NW
    n_chunks = rows_per_w // chunk
    mesh = plsc.VectorSubcoreMesh(
        core_axis_name="c", subcore_axis_name="s", num_cores=NC, num_subcores=NS
    )

    @functools.partial(
        pl.kernel,
        out_type=jax.ShapeDtypeStruct((n_rows, D), jnp.float32),
        mesh=mesh,
        compiler_params=pltpu.CompilerParams(needs_layout_passes=False),
        scratch_types=[
            pltpu.VMEM((num_atoms * D,), jnp.float32),
            pltpu.VMEM((NF, D), jnp.float32),
            pltpu.VMEM((chunk + LANES,), jnp.int32),
            pltpu.VMEM((chunk * NF + LANES,), jnp.float32),
            pltpu.VMEM((chunk, D), jnp.float32),
        ],
    )
    def body(ids_hbm, feat_hbm, tbl_hbm, wfc_hbm, out_hbm, tbl_v, wfc_v, ids_v, feat_v, out_v):
        wid = lax.axis_index("s") * NC + lax.axis_index("c")
        base0 = wid * rows_per_w
        pltpu.sync_copy(tbl_hbm, tbl_v)
        pltpu.sync_copy(wfc_hbm, wfc_v)
        cols = [lax.iota(jnp.int32, LANES) + LANES * j for j in range(D // LANES)]
        w = [
            [wfc_v[k, pl.ds(LANES * j, LANES)] for j in range(D // LANES)]
            for k in range(NF)
        ]

        def chunk_body(it, carry):
            base = base0 + it * chunk
            pltpu.sync_copy(ids_hbm.at[pl.ds(base, chunk)], ids_v.at[pl.ds(0, chunk)])
            pltpu.sync_copy(
                feat_hbm.at[pl.ds(base * NF, chunk * NF)],
                feat_v.at[pl.ds(0, chunk * NF)],
            )

            @plsc.parallel_loop(0, chunk // LANES, unroll=2)
            def group_body(g):
                r0 = g * LANES
                idbase = ids_v[pl.ds(r0, LANES)] * D
                fq = [feat_v[pl.ds(r0 * NF + q * LANES, LANES)] for q in range(4)]
                for rr in range(LANES):
                    f = fq[rr // 4]
                    f0 = f[(rr % 4) * NF + 0]
                    f1 = f[(rr % 4) * NF + 1]
                    f2 = f[(rr % 4) * NF + 2]
                    f3 = f[(rr % 4) * NF + 3]
                    for j in range(D // LANES):
                        g16 = plsc.load_gather(tbl_v, [idbase[rr] + cols[j]])
                        acc = (g16 + (f0 * w[0][j] + f1 * w[1][j])) + (
                            f2 * w[2][j] + f3 * w[3][j]
                        )
                        out_v[r0 + rr, pl.ds(LANES * j, LANES)] = acc
            pltpu.sync_copy(out_v, out_hbm.at[pl.ds(base, chunk)])
            return carry

        lax.fori_loop(0, n_chunks, chunk_body, 0)

    return body(ids, feat, tbl, wfc)


def kernel(atom_ids, atom_features, table, Wf, bf, Wc, bc):
    B, L = atom_ids.shape
    n_rows = B * L
    ids = atom_ids.reshape(n_rows).astype(jnp.int32)
    feat = atom_features.reshape(n_rows * NF)
    bfc = jnp.stack([bf, bc])
    tbl, wfc = _prep(table, Wf, bfc, Wc)
    chunk = 256
    out = _sc_lookup(
        ids, feat, tbl.reshape(-1), wfc,
        num_atoms=table.shape[0], n_rows=n_rows, chunk=chunk,
    )
    return out.reshape(B, L, D)
